# baseline (speedup 1.0000x reference)
import functools

import jax
import jax.numpy as jnp
from jax.experimental import pallas as pl
from jax.experimental.pallas import tpu as pltpu


def _gating_body(x_ref, wg_ref, gates_ref, topi_ref, topg_ref, lb_ref):
    x = x_ref[...]
    wg = wg_ref[...]
    logits = jax.lax.dot_general(
        x, wg, (((1,), (1,)), ((), ())),
        preferred_element_type=jnp.float32)
    n, e = logits.shape
    eio = jax.lax.broadcasted_iota(jnp.int32, (n, e), 1)
    m1 = jnp.max(logits, axis=1, keepdims=True)
    i1 = jnp.min(jnp.where(logits == m1, eio, e), axis=1, keepdims=True)
    mask1 = eio == i1
    l2 = jnp.where(mask1, -jnp.inf, logits)
    m2 = jnp.max(l2, axis=1, keepdims=True)
    i2 = jnp.min(jnp.where(l2 == m2, eio, e), axis=1, keepdims=True)
    mask2 = eio == i2
    ed = jnp.exp(m2 - m1)
    g1 = 1.0 / (1.0 + ed)
    g2 = ed / (1.0 + ed)
    gates = jnp.where(mask1, g1, 0.0) + jnp.where(mask2, g2, 0.0)
    gates_ref[...] = gates
    topi_ref[...] = jnp.concatenate([i1, i2], axis=1)
    topg_ref[...] = jnp.concatenate([g1, g2], axis=1)
    router_prob = jnp.mean(gates, axis=0)
    usage = jnp.mean((gates > 0.0).astype(jnp.float32), axis=0)
    lb_ref[0, 0] = jnp.sum(router_prob * usage) * e


def _gating(xf, Wg):
    n, _ = xf.shape
    e = Wg.shape[0]
    return pl.pallas_call(
        _gating_body,
        out_shape=(
            jax.ShapeDtypeStruct((n, e), jnp.float32),
            jax.ShapeDtypeStruct((n, 2), jnp.int32),
            jax.ShapeDtypeStruct((n, 2), jnp.float32),
            jax.ShapeDtypeStruct((1, 1), jnp.float32),
        ),
        out_specs=(
            pl.BlockSpec(memory_space=pltpu.ANY if False else pltpu.VMEM),
            pl.BlockSpec(memory_space=pltpu.VMEM),
            pl.BlockSpec(memory_space=pltpu.VMEM),
            pl.BlockSpec(memory_space=pltpu.SMEM),
        ),
    )(xf, Wg)


def _expert_body(x_ref, gates_ref, w1_ref, b1_ref, w2_ref, b2_ref,
                 w3_ref, b3_ref, out_ref):
    e = pl.program_id(1)
    x = x_ref[...]
    h1 = jax.lax.dot_general(
        x, w1_ref[0], (((1,), (1,)), ((), ())),
        preferred_element_type=jnp.float32) + b1_ref[0]
    h1 = jnp.maximum(h1, 0.0)
    h2 = jax.lax.dot_general(
        h1, w2_ref[0], (((1,), (1,)), ((), ())),
        preferred_element_type=jnp.float32) + b2_ref[0]
    h2 = jnp.maximum(h2, 0.0)
    o = jax.lax.dot_general(
        h2, w3_ref[0], (((1,), (1,)), ((), ())),
        preferred_element_type=jnp.float32) + b3_ref[0]
    gates = gates_ref[...]
    esel = jax.lax.broadcasted_iota(jnp.int32, gates.shape, 1) == e
    g = jnp.sum(jnp.where(esel, gates, 0.0), axis=1, keepdims=True)
    contrib = o * g

    @pl.when(e == 0)
    def _():
        out_ref[...] = contrib

    @pl.when(e > 0)
    def _():
        out_ref[...] += contrib


def _experts_dense(xf, gates, W1, b1, W2, b2, W3, b3, tile=256):
    n, d = xf.shape
    e, h, _ = W1.shape
    o = W3.shape[1]
    nt = n // tile
    grid = (nt, e)
    return pl.pallas_call(
        _expert_body,
        grid=grid,
        in_specs=[
            pl.BlockSpec((tile, d), lambda i, j: (i, 0)),
            pl.BlockSpec((tile, e), lambda i, j: (i, 0)),
            pl.BlockSpec((1, h, d), lambda i, j: (j, 0, 0)),
            pl.BlockSpec((1, 1, h), lambda i, j: (j, 0, 0)),
            pl.BlockSpec((1, h, h), lambda i, j: (j, 0, 0)),
            pl.BlockSpec((1, 1, h), lambda i, j: (j, 0, 0)),
            pl.BlockSpec((1, o, h), lambda i, j: (j, 0, 0)),
            pl.BlockSpec((1, 1, o), lambda i, j: (j, 0, 0)),
        ],
        out_specs=pl.BlockSpec((tile, o), lambda i, j: (i, 0)),
        out_shape=jax.ShapeDtypeStruct((n, o), jnp.float32),
    )(xf, gates, W1, b1.reshape(e, 1, h), W2, b2.reshape(e, 1, h),
      W3, b3.reshape(e, 1, o))


def kernel(x, Wg, W1, b1, W2, b2, W3, b3):
    bv, sv, dv = x.shape
    xf = x.reshape(-1, dv)
    gates, top_i, top_g, lb = _gating(xf, Wg)
    out = _experts_dense(xf, gates, W1, b1, W2, b2, W3, b3)
    return out.reshape(bv, sv, -1), gates, lb[0, 0]


# speedup vs baseline: 1.5258x; 1.5258x over previous
import functools

import jax
import jax.numpy as jnp
from jax import lax
from jax.experimental import pallas as pl
from jax.experimental.pallas import tpu as pltpu
from jax.experimental.pallas import tpu_sc as plsc

N = 2048
D = 768
H = 1536
O = 768
E = 8
K = 2
P = N * K
T = 256
NT = P // T + E
RPAD = NT * T
LOG2T = 8
NWR = 16
NWG = 32


def _gating_body(x_ref, wg_ref, gates_ref, topi_ref, topg_ref, lb_ref):
    x = x_ref[...]
    wg = wg_ref[...]
    logits = jax.lax.dot_general(
        x, wg, (((1,), (1,)), ((), ())),
        preferred_element_type=jnp.float32)
    n, e = logits.shape
    eio = jax.lax.broadcasted_iota(jnp.int32, (n, e), 1)
    m1 = jnp.max(logits, axis=1, keepdims=True)
    i1 = jnp.min(jnp.where(logits == m1, eio, e), axis=1, keepdims=True)
    mask1 = eio == i1
    l2 = jnp.where(mask1, -jnp.inf, logits)
    m2 = jnp.max(l2, axis=1, keepdims=True)
    i2 = jnp.min(jnp.where(l2 == m2, eio, e), axis=1, keepdims=True)
    mask2 = eio == i2
    ed = jnp.exp(m2 - m1)
    g1 = 1.0 / (1.0 + ed)
    g2 = ed / (1.0 + ed)
    gates = jnp.where(mask1, g1, 0.0) + jnp.where(mask2, g2, 0.0)
    gates_ref[...] = gates
    topi_ref[...] = jnp.concatenate([i1, i2], axis=1)
    topg_ref[...] = jnp.concatenate([g1, g2], axis=1)
    router_prob = jnp.mean(gates, axis=0)
    usage = jnp.mean((gates > 0.0).astype(jnp.float32), axis=0)
    lb_ref[0, 0] = jnp.sum(router_prob * usage) * e


def _gating(xf, Wg):
    n, _ = xf.shape
    e = Wg.shape[0]
    return pl.pallas_call(
        _gating_body,
        out_shape=(
            jax.ShapeDtypeStruct((n, e), jnp.float32),
            jax.ShapeDtypeStruct((n, 2), jnp.int32),
            jax.ShapeDtypeStruct((n, 2), jnp.float32),
            jax.ShapeDtypeStruct((1, 1), jnp.float32),
        ),
        out_specs=(
            pl.BlockSpec(memory_space=pltpu.VMEM),
            pl.BlockSpec(memory_space=pltpu.VMEM),
            pl.BlockSpec(memory_space=pltpu.VMEM),
            pl.BlockSpec(memory_space=pltpu.SMEM),
        ),
    )(xf, Wg)


_PW = P // NWR
_ZW = RPAD // NWR


def _ieq(a, b):
    return jnp.maximum(1 - jnp.abs(a - b), 0)


def _igt(a, b):
    return jnp.minimum(jnp.maximum(a - b, 0), 1)


def _isel(m, a, b):
    return b + m * (a - b)


def _count_body(ei_hbm, cnt_hbm, ei_v, cnt_v):
    c = lax.axis_index("c")
    s = lax.axis_index("s")
    lane = lax.iota(jnp.int32, 16)

    @pl.when(c == 0)
    def _work():
        base_p = s * _PW
        pltpu.sync_copy(ei_hbm.at[pl.ds(base_p, _PW)], ei_v)
        cnt = jnp.zeros((16,), jnp.int32)
        for i in range(_PW // 16):
            ev = ei_v[pl.ds(i * 16, 16)]
            for k in range(16):
                cnt = cnt + _ieq(lane, ev[k])
        cnt_v[...] = cnt
        pltpu.sync_copy(cnt_v, cnt_hbm.at[s])


def _count(e_flat):
    mesh = plsc.VectorSubcoreMesh(core_axis_name="c", subcore_axis_name="s")
    f = functools.partial(
        pl.kernel,
        out_type=jax.ShapeDtypeStruct((NWR, 16), jnp.int32),
        mesh=mesh,
        scratch_types=[
            pltpu.VMEM((_PW,), jnp.int32),
            pltpu.VMEM((16,), jnp.int32),
        ],
    )
    return f(_count_body)(e_flat)


def _route_body(ei_hbm, gv_hbm, x_hbm, cnt_hbm, xs_hbm, gate_hbm, texp_hbm,
                pos_hbm, ei_v, gv_v, dest_v, tok_v, all_v, tex_v, rows_v, sem):
    c = lax.axis_index("c")
    s = lax.axis_index("s")
    lane = lax.iota(jnp.int32, 16)

    @pl.when(c == 0)
    def _work():
        base_p = s * _PW
        pltpu.sync_copy(ei_hbm.at[pl.ds(base_p, _PW)], ei_v)
        pltpu.sync_copy(gv_hbm.at[pl.ds(base_p, 128)], gv_v.at[0])
        pltpu.sync_copy(gv_hbm.at[pl.ds(base_p + 128, 128)], gv_v.at[1])
        pltpu.sync_copy(cnt_hbm, all_v)
        tot = jnp.zeros((16,), jnp.int32)
        pb = jnp.zeros((16,), jnp.int32)
        for w in range(NWR):
            row = all_v[w]
            tot = tot + row
            pb = pb + _igt(s, w) * row
        padded = ((tot + (T - 1)) >> LOG2T) << LOG2T
        off = jnp.zeros((16,), jnp.int32)
        for k in range(E):
            off = off + _igt(lane, k) * padded[k]
        base = off + pb

        @pl.when(s == 0)
        def _meta():
            total = padded[0]
            for k in range(1, E):
                total = total + padded[k]
            live = total >> LOG2T
            for cc in range(2):
                tv = lane + cc * 16
                srow = jnp.minimum(tv << LOG2T, total - 1)
                tex = jnp.zeros((16,), jnp.int32)
                for j in range(E):
                    oj = off[j]
                    pj = padded[j]
                    inb = _igt(srow, oj - 1) * _igt(oj + pj, srow)
                    tex = _isel(inb, j, tex)
                tex = _isel(_ieq(tv, NT), live, tex)
                tex_v[pl.ds(cc * 16, 16)] = tex
            pltpu.sync_copy(tex_v, texp_hbm)

        for i in range(_PW // 16):
            ev = ei_v[pl.ds(i * 16, 16)]
            rank = jnp.zeros((16,), jnp.int32)
            hist = jnp.zeros((16,), jnp.int32)
            for k in range(16):
                evk = ev[k]
                if k < 15:
                    rank = rank + _igt(lane, k) * _ieq(ev, evk)
                hist = hist + _ieq(lane, evk)
            dest = jnp.zeros((16,), jnp.int32)
            for j in range(E):
                dest = _isel(_ieq(ev, j), base[j] + rank, dest)
            base = base + hist
            dest_v[i // 8, pl.ds((i % 8) * 16, 16)] = dest
            tok_v[i // 8, pl.ds((i % 8) * 16, 16)] = (base_p + i * 16 + lane) >> 1
        for j in range(2):
            pltpu.async_copy(x_hbm.at[tok_v.at[j]], rows_v, sem).wait()
            pltpu.async_copy(rows_v, xs_hbm.at[dest_v.at[j]], sem).wait()
        pltpu.async_copy(gv_v.at[0], gate_hbm.at[dest_v.at[0]], sem).wait()
        pltpu.async_copy(gv_v.at[1], gate_hbm.at[dest_v.at[1]], sem).wait()
        pltpu.sync_copy(dest_v, pos_hbm.at[pl.ds(s * 2, 2)])


def _route(e_flat, g_flat, xf):
    cnt_all = _count(e_flat)
    mesh = plsc.VectorSubcoreMesh(core_axis_name="c", subcore_axis_name="s")
    f = functools.partial(
        pl.kernel,
        out_type=(
            jax.ShapeDtypeStruct((RPAD, D), jnp.float32),
            jax.ShapeDtypeStruct((RPAD,), jnp.float32),
            jax.ShapeDtypeStruct((NT + 8,), jnp.int32),
            jax.ShapeDtypeStruct((NWR * 2, 128), jnp.int32),
        ),
        mesh=mesh,
        scratch_types=[
            pltpu.VMEM((_PW,), jnp.int32),
            pltpu.VMEM((2, 128), jnp.float32),
            pltpu.VMEM((2, 128), jnp.int32),
            pltpu.VMEM((2, 128), jnp.int32),
            pltpu.VMEM((NWR, 16), jnp.int32),
            pltpu.VMEM((32,), jnp.int32),
            pltpu.VMEM((128, D), jnp.float32),
            pltpu.SemaphoreType.DMA,
        ],
    )
    return f(_route_body)(e_flat, g_flat, xf, cnt_all)


def _ggemm_body(meta_ref, xs_ref, gp_ref, w1_ref, b1_ref, w2_ref, b2_ref,
                w3_ref, b3_ref, o_ref):
    t = pl.program_id(0)
    live = meta_ref[NT]

    @pl.when(t < live)
    def _():
        x = xs_ref[...].astype(jnp.bfloat16)
        h1 = jax.lax.dot_general(
            x, w1_ref[0], (((1,), (1,)), ((), ())),
            preferred_element_type=jnp.float32) + b1_ref[0]
        h1 = jnp.maximum(h1, 0.0).astype(jnp.bfloat16)
        h2 = jax.lax.dot_general(
            h1, w2_ref[0], (((1,), (1,)), ((), ())),
            preferred_element_type=jnp.float32) + b2_ref[0]
        h2 = jnp.maximum(h2, 0.0).astype(jnp.bfloat16)
        o = jax.lax.dot_general(
            h2, w3_ref[0], (((1,), (1,)), ((), ())),
            preferred_element_type=jnp.float32) + b3_ref[0]
        o_ref[...] = o * gp_ref[...]

    @pl.when(t >= live)
    def _():
        o_ref[...] = jnp.zeros_like(o_ref)


def _ggemm(meta, xs, gp, W1, b1, W2, b2, W3, b3):
    grid_spec = pltpu.PrefetchScalarGridSpec(
        num_scalar_prefetch=1,
        grid=(NT,),
        in_specs=[
            pl.BlockSpec((T, D), lambda t, m: (t, 0)),
            pl.BlockSpec((T, 1), lambda t, m: (t, 0)),
            pl.BlockSpec((1, H, D), lambda t, m: (m[t], 0, 0)),
            pl.BlockSpec((1, 1, H), lambda t, m: (m[t], 0, 0)),
            pl.BlockSpec((1, H, H), lambda t, m: (m[t], 0, 0)),
            pl.BlockSpec((1, 1, H), lambda t, m: (m[t], 0, 0)),
            pl.BlockSpec((1, O, H), lambda t, m: (m[t], 0, 0)),
            pl.BlockSpec((1, 1, O), lambda t, m: (m[t], 0, 0)),
        ],
        out_specs=pl.BlockSpec((T, O), lambda t, m: (t, 0)),
    )
    return pl.pallas_call(
        _ggemm_body,
        grid_spec=grid_spec,
        out_shape=jax.ShapeDtypeStruct((RPAD, O), jnp.float32),
    )(meta, xs, gp.reshape(RPAD, 1),
      W1.astype(jnp.bfloat16), b1.reshape(E, 1, H),
      W2.astype(jnp.bfloat16), b2.reshape(E, 1, H),
      W3.astype(jnp.bfloat16), b3.reshape(E, 1, O))


_TW = N // NWG
_CC = 32


def _combine_body(pos_hbm, o_hbm, out_hbm, idx_v, rows_v, out_v, sem):
    w = lax.axis_index("s") * 2 + lax.axis_index("c")
    for ch in range(_TW // _CC):
        t0 = w * _TW + ch * _CC
        pltpu.sync_copy(pos_hbm.at[pl.ds(t0 * K, _CC * K)], idx_v)
        pltpu.async_copy(o_hbm.at[idx_v], rows_v, sem).wait()

        def body(t, carry):
            for cc in range(O // 16):
                s0 = rows_v[2 * t, pl.ds(cc * 16, 16)]
                s1 = rows_v[2 * t + 1, pl.ds(cc * 16, 16)]
                out_v[t, pl.ds(cc * 16, 16)] = s0 + s1
            return carry

        lax.fori_loop(0, _CC, body, 0)
        pltpu.sync_copy(out_v, out_hbm.at[pl.ds(t0, _CC)])


def _combine(pos_flat, o):
    mesh = plsc.VectorSubcoreMesh(core_axis_name="c", subcore_axis_name="s")
    f = functools.partial(
        pl.kernel,
        out_type=jax.ShapeDtypeStruct((N, O), jnp.float32),
        mesh=mesh,
        scratch_types=[
            pltpu.VMEM((_CC * K,), jnp.int32),
            pltpu.VMEM((_CC * K, O), jnp.float32),
            pltpu.VMEM((_CC, O), jnp.float32),
            pltpu.SemaphoreType.DMA,
        ],
    )
    return f(_combine_body)(pos_flat, o)


def kernel(x, Wg, W1, b1, W2, b2, W3, b3):
    bv, sv, dv = x.shape
    xf = x.reshape(-1, dv)
    gates, top_i, top_g, lb = _gating(xf, Wg)
    xs, gate_pad, meta, pos = _route(top_i.reshape(-1), top_g.reshape(-1), xf)
    o = _ggemm(meta, xs, gate_pad, W1, b1, W2, b2, W3, b3)
    out = _combine(pos.reshape(-1), o)
    return out.reshape(bv, sv, -1), gates, lb[0, 0]
